# baseline (speedup 1.0000x reference)
import functools
import jax
import jax.numpy as jnp
from jax import lax
from jax.experimental import pallas as pl
from jax.experimental.pallas import tpu as pltpu
from jax.experimental.pallas import tpu_sc as plsc

_B, _S, _D = 2, 2048, 1024
_E = 8
_F = 1024
_ACTIVE_K = 8
_BS = _B * _S
_NSEL = _B * _ACTIVE_K

_TSG = 1024
_NGG = _BS // _TSG
_HC = 512
_CPE = 4
_NES = _E * _CPE
_TSO = 512
_NSC = _BS // _TSO
_RING = 8
_LEAD = 4
_NW = 32
_CHUNK = _BS // _NW
_BIG = 1 << 30


def _top2_gs(logits):
    m = jnp.max(logits, axis=-1, keepdims=True)
    p = jnp.exp(logits - m)
    p = p / jnp.sum(p, axis=-1, keepdims=True)
    eidx = jax.lax.broadcasted_iota(jnp.int32, p.shape, 1)
    m1 = jnp.max(p, axis=-1, keepdims=True)
    i1 = jnp.min(jnp.where(p == m1, eidx, _E), axis=-1, keepdims=True)
    p2 = jnp.where(eidx == i1, -jnp.inf, p)
    m2 = jnp.max(p2, axis=-1, keepdims=True)
    i2 = jnp.min(jnp.where(p2 == m2, eidx, _E), axis=-1, keepdims=True)
    mask = (eidx == i1) | (eidx == i2)
    gs = jnp.where(mask, p, 0.0)
    return gs / (m1 + m2 + 1e-9)


def _gate_body(x_ref, gw_ref, gb_ref, lw_ref, lb_ref, gs_ref, ls_ref):
    xt = x_ref[...]
    logits = jnp.dot(xt, gw_ref[...], preferred_element_type=jnp.float32)
    gs_ref[...] = _top2_gs(logits + gb_ref[...])
    ls = jnp.dot(xt, lw_ref[...], preferred_element_type=jnp.float32)
    ls_ref[...] = ls + lb_ref[...]


_sc_mesh = plsc.VectorSubcoreMesh(core_axis_name="c", subcore_axis_name="s")

_GDIMS = lax.GatherDimensionNumbers(
    offset_dims=(), collapsed_slice_dims=(0,), start_index_map=(0,))


def _shuf(v, perm):
    return lax.gather(v, perm[:, None], _GDIMS, (1,),
                      mode=lax.GatherScatterMode.PROMISE_IN_BOUNDS)


def _bmax(v):
    lane = lax.iota(jnp.int32, 16)
    for sh in (8, 4, 2, 1):
        v = jnp.maximum(v, _shuf(v, lane ^ sh))
    return v


def _bmin_i32(v):
    return -_bmax(-v)


@functools.partial(
    pl.kernel,
    mesh=_sc_mesh,
    out_type=[
        jax.ShapeDtypeStruct((_NSEL,), jnp.int32),
        jax.ShapeDtypeStruct((_NSEL, _D), jnp.float32),
    ],
    scratch_types=[
        pltpu.VMEM((_CHUNK,), jnp.float32),
        pltpu.VMEM((16,), jnp.float32),
        pltpu.VMEM((16,), jnp.int32),
        pltpu.VMEM((16 * 16,), jnp.float32),
        pltpu.VMEM((16 * 16,), jnp.int32),
        pltpu.VMEM((16,), jnp.int32),
        pltpu.VMEM((16,), jnp.int32),
        pltpu.VMEM((16,), jnp.int32),
        pltpu.VMEM((_NSEL, _D), jnp.float32),
        pltpu.VMEM_SHARED((_NW * 16,), jnp.float32),
        pltpu.VMEM_SHARED((_NW * 16,), jnp.int32),
        pltpu.VMEM_SHARED((2 * 16,), jnp.int32),
        pltpu.SemaphoreType.DMA,
    ],
)
def _sc_route(ls_hbm, x_hbm, gidx_out, xsel_out,
              ls_loc, lval_v, lidx_v, mvals, midx, iv0_v, iv1_v, fidx_v,
              xsel_v, sval, sidxs, sres, sem):
    wid = lax.axis_index("s") * 2 + lax.axis_index("c")
    lane = lax.iota(jnp.int32, 16)
    base = wid * _CHUNK

    pltpu.sync_copy(ls_hbm.at[pl.ds(base, _CHUNK)], ls_loc)
    valv = jnp.full((16,), -jnp.inf, jnp.float32)
    idxv = jnp.full((16,), _BIG, jnp.int32)
    for k in range(_ACTIVE_K):
        def sbody(j, carry):
            m16, i16 = carry
            v = ls_loc[pl.ds(j * 16, 16)]
            gi = base + j * 16 + lane
            upd = v > m16
            return jnp.where(upd, v, m16), jnp.where(upd, gi, i16)
        m16, i16 = lax.fori_loop(
            0, _CHUNK // 16, sbody,
            (jnp.full((16,), -jnp.inf, jnp.float32),
             jnp.full((16,), _BIG, jnp.int32)))
        m_b = _bmax(m16)
        gsel_b = _bmin_i32(jnp.where(m16 == m_b, i16, _BIG))
        valv = jnp.where(lane == k, m_b, valv)
        idxv = jnp.where(lane == k, gsel_b, idxv)

        def clr(j, carry):
            v = ls_loc[pl.ds(j * 16, 16)]
            gi = base + j * 16 + lane
            ls_loc[pl.ds(j * 16, 16)] = jnp.where(gi == gsel_b, -jnp.inf, v)
            return carry
        lax.fori_loop(0, _CHUNK // 16, clr, 0)
    lval_v[...] = valv
    lidx_v[...] = idxv
    pltpu.sync_copy(lval_v, sval.at[pl.ds(wid * 16, 16)])
    pltpu.sync_copy(lidx_v, sidxs.at[pl.ds(wid * 16, 16)])
    plsc.subcore_barrier()

    @pl.when(wid < _B)
    def _():
        pltpu.sync_copy(sval.at[pl.ds(wid * 256, 256)], mvals)
        pltpu.sync_copy(sidxs.at[pl.ds(wid * 256, 256)], midx)
        res = jnp.full((16,), _BIG, jnp.int32)
        for k in range(_ACTIVE_K):
            def mbody(r, carry):
                m16, i16 = carry
                v = mvals[pl.ds(r * 16, 16)]
                vi = midx[pl.ds(r * 16, 16)]
                upd = (v > m16) | ((v == m16) & (vi < i16))
                return jnp.where(upd, v, m16), jnp.where(upd, vi, i16)
            m16, i16 = lax.fori_loop(
                0, 16, mbody,
                (jnp.full((16,), -jnp.inf, jnp.float32),
                 jnp.full((16,), _BIG, jnp.int32)))
            m_b = _bmax(m16)
            gsel_b = _bmin_i32(jnp.where(m16 == m_b, i16, _BIG))
            res = jnp.where(lane == wid * _ACTIVE_K + k, gsel_b, res)

            def rbody(r, carry):
                v = mvals[pl.ds(r * 16, 16)]
                vi = midx[pl.ds(r * 16, 16)]
                mvals[pl.ds(r * 16, 16)] = jnp.where(vi == gsel_b, -jnp.inf, v)
                return carry
            lax.fori_loop(0, 16, rbody, 0)
        lidx_v[...] = res
        pltpu.sync_copy(lidx_v, sres.at[pl.ds(wid * 16, 16)])
    plsc.subcore_barrier()

    @pl.when(wid == 0)
    def _():
        pltpu.sync_copy(sres.at[pl.ds(0, 16)], iv0_v)
        pltpu.sync_copy(sres.at[pl.ds(16, 16)], iv1_v)
        fidx_v[...] = jnp.where(lane < _ACTIVE_K, iv0_v[...], iv1_v[...])
        pltpu.sync_copy(fidx_v, gidx_out)
        pltpu.async_copy(x_hbm.at[fidx_v], xsel_v, sem).wait()
        pltpu.sync_copy(xsel_v, xsel_out)


def _issue(i, op, w1_hbm, w2_hbm, wring, sem):
    e = i // _CPE
    c = jax.lax.rem(i, _CPE)
    slot = jax.lax.rem(i, _RING)

    @pl.when(c < 2)
    def _():
        cp = pltpu.make_async_copy(
            w1_hbm.at[e, pl.ds(c * _HC, _HC), :], wring.at[slot], sem.at[slot])
        cp.start() if op == "start" else cp.wait()

    @pl.when(c >= 2)
    def _():
        cp = pltpu.make_async_copy(
            w2_hbm.at[e, pl.ds((c - 2) * _HC, _HC), :], wring.at[slot],
            sem.at[slot])
        cp.start() if op == "start" else cp.wait()


def _exp_body(xsel_ref, gidx_ref, gw_ref, gb_ref,
              w1_hbm, b1_ref, w2_hbm, b2_ref,
              out_ref, gsel_s, osel_s, hpre_s, hrelu_s, wring, sem):
    g = pl.program_id(0)

    @pl.when(g == 0)
    def _prime():
        for i in range(_LEAD):
            _issue(i, "start", w1_hbm, w2_hbm, wring, sem)
        lg = jnp.dot(xsel_ref[...], gw_ref[...],
                     preferred_element_type=jnp.float32)
        gsel_s[...] = _top2_gs(lg + gb_ref[...])

    @pl.when((g >= 1) & (g + _LEAD - 1 < _NES))
    def _prefetch():
        _issue(g + _LEAD - 1, "start", w1_hbm, w2_hbm, wring, sem)

    @pl.when(g < _NES)
    def _expert():
        j = g
        e = j // _CPE
        c = jax.lax.rem(j, _CPE)
        slot = jax.lax.rem(j, _RING)
        _issue(j, "wait", w1_hbm, w2_hbm, wring, sem)
        wv = wring[pl.ds(slot, 1)][0]

        @pl.when(c == 0)
        def _():
            hpre_s[...] = jnp.dot(xsel_ref[:, 0:_HC], wv,
                                  preferred_element_type=jnp.float32)

        @pl.when(c == 1)
        def _():
            hpre_s[...] += jnp.dot(xsel_ref[:, _HC:_D], wv,
                                   preferred_element_type=jnp.float32)

        gss = gsel_s[...]
        eidx = jax.lax.broadcasted_iota(jnp.int32, gss.shape, 1)
        gcol = jnp.sum(jnp.where(eidx == e, gss, 0.0), axis=1, keepdims=True)

        @pl.when(c == 2)
        def _():
            b1v = b1_ref[pl.ds(e, 1)][0]
            hr = jnp.maximum(hpre_s[...] + b1v, 0.0)
            hrelu_s[...] = hr

            @pl.when(j == 2)
            def _():
                osel_s[...] = jnp.zeros_like(osel_s)

            b2v = b2_ref[pl.ds(e, 1)][0]
            osel_s[...] += gcol * (
                jnp.dot(hr[:, 0:_HC], wv, preferred_element_type=jnp.float32)
                + b2v)

        @pl.when(c == 3)
        def _():
            osel_s[...] += gcol * jnp.dot(
                hrelu_s[:, _HC:_F], wv, preferred_element_type=jnp.float32)

    @pl.when(g >= _NES)
    def _scatter():
        t = g - _NES
        row = (t * _TSO
               + jax.lax.broadcasted_iota(jnp.int32, (_TSO, _NSEL), 0))
        onehot = (row == gidx_ref[...]).astype(jnp.float32)
        out_ref[...] = jnp.dot(onehot, osel_s[...],
                               preferred_element_type=jnp.float32)


@jax.jit
def kernel(x, gate_w, gate_b, local_w, local_b, W1, b1, W2, b2):
    xf = x.reshape(_BS, _D)
    gb2 = gate_b.reshape(1, _E)
    lb2 = local_b.reshape(1, 1)

    gs_flat, ls_flat = pl.pallas_call(
        _gate_body,
        grid=(_NGG,),
        in_specs=[
            pl.BlockSpec((_TSG, _D), lambda g: (g, 0)),
            pl.BlockSpec((_D, _E), lambda g: (0, 0)),
            pl.BlockSpec((1, _E), lambda g: (0, 0)),
            pl.BlockSpec((_D, 1), lambda g: (0, 0)),
            pl.BlockSpec((1, 1), lambda g: (0, 0)),
        ],
        out_specs=[
            pl.BlockSpec((_TSG, _E), lambda g: (g, 0)),
            pl.BlockSpec((_TSG, 1), lambda g: (g, 0)),
        ],
        out_shape=[
            jax.ShapeDtypeStruct((_BS, _E), jnp.float32),
            jax.ShapeDtypeStruct((_BS, 1), jnp.float32),
        ],
    )(xf, gate_w, gb2, local_w, lb2)

    gidx, x_sel = _sc_route(ls_flat.reshape(_BS), xf)

    out_flat = pl.pallas_call(
        _exp_body,
        grid=(_NES + _NSC,),
        in_specs=[
            pl.BlockSpec((_NSEL, _D), lambda g: (0, 0)),
            pl.BlockSpec((1, _NSEL), lambda g: (0, 0)),
            pl.BlockSpec((_D, _E), lambda g: (0, 0)),
            pl.BlockSpec((1, _E), lambda g: (0, 0)),
            pl.BlockSpec(memory_space=pl.ANY),
            pl.BlockSpec((_E, 1, _F), lambda g: (0, 0, 0)),
            pl.BlockSpec(memory_space=pl.ANY),
            pl.BlockSpec((_E, 1, _F), lambda g: (0, 0, 0)),
        ],
        out_specs=pl.BlockSpec((_TSO, _F),
                               lambda g: (jnp.clip(g - _NES, 0, _NSC - 1), 0)),
        out_shape=jax.ShapeDtypeStruct((_BS, _F), jnp.float32),
        scratch_shapes=[
            pltpu.VMEM((_NSEL, _E), jnp.float32),
            pltpu.VMEM((_NSEL, _F), jnp.float32),
            pltpu.VMEM((_NSEL, _F), jnp.float32),
            pltpu.VMEM((_NSEL, _F), jnp.float32),
            pltpu.VMEM((_RING, _HC, _F), jnp.float32),
            pltpu.SemaphoreType.DMA((_RING,)),
        ],
    )(x_sel, gidx.reshape(1, _NSEL), gate_w, gb2,
      W1, b1.reshape(_E, 1, _F), W2, b2.reshape(_E, 1, _F))

    return out_flat.reshape(_B, _S, _F), gs_flat.reshape(_B, _S, _E)
